# baseline (speedup 1.0000x reference)
import functools

import jax
import jax.numpy as jnp
from jax import lax
from jax.experimental import pallas as pl
from jax.experimental.pallas import tpu as pltpu
from jax.experimental.pallas import tpu_sc as plsc

NC = 2
NS = 16
NW = NC * NS
C = 128
NBUF = 10
LAG = 2


def _make_gather(V, D, B):
    b_per_w = B // NW
    n_chunks = b_per_w // C
    mesh = plsc.VectorSubcoreMesh(core_axis_name="c", subcore_axis_name="s")

    @functools.partial(
        pl.kernel,
        mesh=mesh,
        out_type=jax.ShapeDtypeStruct((B, D), jnp.float32),
        compiler_params=pltpu.CompilerParams(use_tc_tiling_on_sc=False),
        scratch_types=[
            pltpu.VMEM((n_chunks, C), jnp.int32),
            pltpu.VMEM((NBUF, C, D), jnp.float32),
            pltpu.SemaphoreType.DMA((NBUF,)),
            pltpu.SemaphoreType.DMA((NBUF,)),
        ],
    )
    def k(table_hbm, idx_hbm, out_hbm, idx_v, rows_v, gsem, osem):
        wid = lax.axis_index("s") * NC + lax.axis_index("c")
        base = wid * b_per_w
        pltpu.sync_copy(idx_hbm.at[wid], idx_v)

        def issue_gather(g, b):
            pltpu.async_copy(table_hbm.at[idx_v.at[g]], rows_v.at[b],
                             gsem.at[b])

        def wait_gather(b):
            pltpu.make_async_copy(table_hbm.at[pl.ds(0, C)], rows_v.at[b],
                                  gsem.at[b]).wait()

        def wait_write(b):
            pltpu.make_async_copy(table_hbm.at[pl.ds(0, C)], rows_v.at[b],
                                  osem.at[b]).wait()

        for b in range(NBUF):
            issue_gather(b, b)

        def body(t, carry):
            for b in range(NBUF):
                g = t * NBUF + b
                wait_gather(b)
                pltpu.async_copy(rows_v.at[b],
                                 out_hbm.at[pl.ds(base + g * C, C)],
                                 osem.at[b])
                b2 = (b + NBUF - LAG) % NBUF
                g2 = g + NBUF - LAG

                @pl.when(jnp.logical_and(g >= LAG, g2 < n_chunks))
                def _():
                    wait_write(b2)
                    issue_gather(g2, b2)

            return carry

        lax.fori_loop(0, n_chunks // NBUF, body, 0)

        for b in range(NBUF):
            wait_write(b)

    return k


def kernel(x, embeddings):
    Bx, H = x.shape
    V, D = embeddings.shape
    B = Bx * H
    idx = x.reshape(NW, (B // NW) // C, C).astype(jnp.int32)
    out = _make_gather(V, D, B)(embeddings, idx)
    return out.reshape(Bx, H, D)


# speedup vs baseline: 1.0046x; 1.0046x over previous
import functools

import jax
import jax.numpy as jnp
from jax import lax
from jax.experimental import pallas as pl
from jax.experimental.pallas import tpu as pltpu
from jax.experimental.pallas import tpu_sc as plsc

NC = 2
NS = 16
NW = NC * NS
L = 16
C = 128
NBUF = 10
LAG = 2


def _make_gather(V, D, B):
    b_per_w = B // NW
    n_chunks = b_per_w // C
    mesh = plsc.VectorSubcoreMesh(core_axis_name="c", subcore_axis_name="s")

    @functools.partial(
        pl.kernel,
        mesh=mesh,
        out_type=jax.ShapeDtypeStruct((B, D), jnp.float32),
        compiler_params=pltpu.CompilerParams(use_tc_tiling_on_sc=False),
        scratch_types=[
            pltpu.VMEM((n_chunks, C), jnp.int32),
            pltpu.VMEM((NBUF, C, D), jnp.float32),
            pltpu.SemaphoreType.DMA((NBUF,)),
            pltpu.SemaphoreType.DMA((NBUF,)),
        ],
    )
    def k(table_hbm, idx_hbm, out_hbm, idx_v, rows_v, gsem, osem):
        wid = lax.axis_index("s") * NC + lax.axis_index("c")
        base = wid * b_per_w
        pltpu.sync_copy(idx_hbm.at[wid], idx_v)

        def issue_gathers(g, b):
            for v in range(C // L):
                iv = idx_v[g, pl.ds(v * L, L)]
                pltpu.async_copy(table_hbm.at[iv],
                                 rows_v.at[b].at[pl.ds(v * L, L)],
                                 gsem.at[b])

        def wait_gathers(b):
            for v in range(C // L):
                pltpu.make_async_copy(table_hbm.at[pl.ds(0, L)],
                                      rows_v.at[b].at[pl.ds(v * L, L)],
                                      gsem.at[b]).wait()

        def wait_write(b):
            pltpu.make_async_copy(table_hbm.at[pl.ds(0, C)], rows_v.at[b],
                                  osem.at[b]).wait()

        for b in range(NBUF):
            issue_gathers(b, b)

        def body(t, carry):
            for b in range(NBUF):
                g = t * NBUF + b
                wait_gathers(b)
                pltpu.async_copy(rows_v.at[b],
                                 out_hbm.at[pl.ds(base + g * C, C)],
                                 osem.at[b])
                b2 = (b + NBUF - LAG) % NBUF
                g2 = g + NBUF - LAG

                @pl.when(jnp.logical_and(g >= LAG, g2 < n_chunks))
                def _():
                    wait_write(b2)
                    issue_gathers(g2, b2)

            return carry

        lax.fori_loop(0, n_chunks // NBUF, body, 0)

        for b in range(NBUF):
            wait_write(b)

    return k


def kernel(x, embeddings):
    Bx, H = x.shape
    V, D = embeddings.shape
    B = Bx * H
    idx = x.reshape(NW, (B // NW) // C, C).astype(jnp.int32)
    out = _make_gather(V, D, B)(embeddings, idx)
    return out.reshape(Bx, H, D)
